# baseline (speedup 1.0000x reference)
import functools

import jax
import jax.numpy as jnp
from jax import lax
from jax.experimental import pallas as pl
from jax.experimental.pallas import tpu as pltpu
from jax.experimental.pallas import tpu_sc as plsc


def _sc_gather(uid, iid, ut, it):
    B = uid.shape[0]
    D = ut.shape[1]
    info = plsc.get_sparse_core_info()
    NC, NS = info.num_cores, info.num_subcores
    NW = NC * NS
    BPW = B // NW
    CH = 128

    mesh = plsc.VectorSubcoreMesh(core_axis_name="c", subcore_axis_name="s")

    @functools.partial(
        pl.kernel,
        mesh=mesh,
        compiler_params=pltpu.CompilerParams(use_tc_tiling_on_sc=False),
        out_type=[jax.ShapeDtypeStruct((B, D), jnp.float32)] * 2,
        scratch_types=[
            pltpu.VMEM((BPW,), jnp.int32),
            pltpu.VMEM((BPW,), jnp.int32),
            pltpu.VMEM((BPW, D), jnp.float32),
            pltpu.VMEM((BPW, D), jnp.float32),
            pltpu.SemaphoreType.DMA,
        ],
    )
    def gather_k(uid_hbm, iid_hbm, ut_hbm, it_hbm,
                 o_u, o_i,
                 uid_v, iid_v, bu, bi, sem):
        wid = lax.axis_index("s") * NC + lax.axis_index("c")
        base = wid * BPW
        pltpu.sync_copy(uid_hbm.at[pl.ds(base, BPW)], uid_v)
        pltpu.sync_copy(iid_hbm.at[pl.ds(base, BPW)], iid_v)
        copies = []
        for c in range(BPW // CH):
            s = pl.ds(c * CH, CH)
            copies.append(pltpu.async_copy(ut_hbm.at[uid_v.at[s]], bu.at[s], sem))
            copies.append(pltpu.async_copy(it_hbm.at[iid_v.at[s]], bi.at[s], sem))
        for cp in copies:
            cp.wait()
        pltpu.sync_copy(bu, o_u.at[pl.ds(base, BPW)])
        pltpu.sync_copy(bi, o_i.at[pl.ds(base, BPW)])

    return gather_k(uid, iid, ut, it)


def _dense_body(u_r, i_r, w1a_r, w1b_r, b1_r, w2_r, b2_r,
                w3_r, b3_r, w4_r, b4_r, wg_r, wm_r, bo_r, o_r):
    f32 = jnp.float32
    P = w1a_r.shape[0]
    mu = u_r[:, :P]
    mi = i_r[:, :P]
    g = u_r[:, P:2 * P] * i_r[:, P:2 * P]
    h = jnp.maximum(
        jnp.dot(mu, w1a_r[...], preferred_element_type=f32)
        + jnp.dot(mi, w1b_r[...], preferred_element_type=f32)
        + b1_r[...], 0.0)
    h = jnp.maximum(jnp.dot(h, w2_r[...], preferred_element_type=f32) + b2_r[...], 0.0)
    h = jnp.maximum(jnp.dot(h, w3_r[...], preferred_element_type=f32) + b3_r[...], 0.0)
    h = jnp.maximum(jnp.dot(h, w4_r[...], preferred_element_type=f32) + b4_r[...], 0.0)
    logit = (jnp.dot(g, wg_r[...], preferred_element_type=f32)
             + jnp.dot(h, wm_r[...], preferred_element_type=f32)
             + bo_r[...])
    o_r[...] = jax.nn.sigmoid(logit)[:, 0]


def _dense(u, i, w1a, w1b, b1, w2, b2, w3, b3, w4, b4, wg, wm, bo):
    B, D = u.shape
    BLK = 16384
    grid = (B // BLK,)

    def rowblk():
        return pl.BlockSpec((BLK, D), lambda k: (k, 0))

    def full(a):
        return pl.BlockSpec(a.shape, lambda k: (0,) * a.ndim)

    return pl.pallas_call(
        _dense_body,
        grid=grid,
        in_specs=[rowblk(), rowblk(),
                  full(w1a), full(w1b), full(b1), full(w2), full(b2),
                  full(w3), full(b3), full(w4), full(b4),
                  full(wg), full(wm), full(bo)],
        out_specs=pl.BlockSpec((BLK,), lambda k: (k,)),
        out_shape=jax.ShapeDtypeStruct((B,), jnp.float32),
    )(u, i, w1a, w1b, b1, w2, b2, w3, b3, w4, b4, wg, wm, bo)


def kernel(x, mlp_user_emb, mlp_item_emb, gmf_user_emb, gmf_item_emb,
           W1, b1, W2, b2, W3, b3, W4, b4, Wout, bout):
    V, P = mlp_user_emb.shape
    ut = jnp.concatenate([mlp_user_emb, gmf_user_emb], axis=1)
    it = jnp.concatenate([mlp_item_emb, gmf_item_emb], axis=1)
    uid = x[:, 0]
    iid = x[:, 1]
    gu, gi = _sc_gather(uid, iid, ut, it)
    return _dense(gu, gi,
                  W1[:P], W1[P:], b1[None, :], W2, b2[None, :],
                  W3, b3[None, :], W4, b4[None, :],
                  Wout[:P], Wout[P:], bout[None, :])


# speedup vs baseline: 1.0148x; 1.0148x over previous
import functools

import jax
import jax.numpy as jnp
from jax import lax
from jax.experimental import pallas as pl
from jax.experimental.pallas import tpu as pltpu
from jax.experimental.pallas import tpu_sc as plsc


def _sc_gather(uid, iid, ut, it):
    B = uid.shape[0]
    D = ut.shape[1]
    info = plsc.get_sparse_core_info()
    NC, NS = info.num_cores, info.num_subcores
    NW = NC * NS
    BPW = B // NW
    CH = 128

    mesh = plsc.VectorSubcoreMesh(core_axis_name="c", subcore_axis_name="s")

    @functools.partial(
        pl.kernel,
        mesh=mesh,
        compiler_params=pltpu.CompilerParams(use_tc_tiling_on_sc=False),
        out_type=[jax.ShapeDtypeStruct((B, D), jnp.float32)] * 2,
        scratch_types=[
            pltpu.VMEM((BPW,), jnp.int32),
            pltpu.VMEM((BPW,), jnp.int32),
            pltpu.VMEM((BPW, D), jnp.float32),
            pltpu.VMEM((BPW, D), jnp.float32),
            pltpu.SemaphoreType.DMA,
        ],
    )
    def gather_k(uid_hbm, iid_hbm, ut_hbm, it_hbm,
                 o_u, o_i,
                 uid_v, iid_v, bu, bi, sem):
        wid = lax.axis_index("s") * NC + lax.axis_index("c")
        base = wid * BPW
        pltpu.sync_copy(uid_hbm.at[pl.ds(base, BPW)], uid_v)
        pltpu.sync_copy(iid_hbm.at[pl.ds(base, BPW)], iid_v)
        copies = []
        for c in range(BPW // CH):
            s = pl.ds(c * CH, CH)
            copies.append(pltpu.async_copy(ut_hbm.at[uid_v.at[s]], bu.at[s], sem))
            copies.append(pltpu.async_copy(it_hbm.at[iid_v.at[s]], bi.at[s], sem))
        for cp in copies:
            cp.wait()
        pltpu.sync_copy(bu, o_u.at[pl.ds(base, BPW)])
        pltpu.sync_copy(bi, o_i.at[pl.ds(base, BPW)])

    return gather_k(uid, iid, ut, it)


def _dense_body(u_r, i_r, w1a_r, w1b_r, b1_r, w2_r, b2_r,
                w3_r, b3_r, w4_r, b4_r, wg_r, wm_r, bo_r, o_r):
    f32 = jnp.float32
    P = w1a_r.shape[0]
    mu = u_r[:, :P]
    mi = i_r[:, :P]
    g = u_r[:, P:2 * P] * i_r[:, P:2 * P]
    h = jnp.maximum(
        jnp.dot(mu, w1a_r[...], preferred_element_type=f32)
        + jnp.dot(mi, w1b_r[...], preferred_element_type=f32)
        + b1_r[...], 0.0)
    h = jnp.maximum(jnp.dot(h, w2_r[...], preferred_element_type=f32) + b2_r[...], 0.0)
    h = jnp.maximum(jnp.dot(h, w3_r[...], preferred_element_type=f32) + b3_r[...], 0.0)
    h = jnp.maximum(jnp.dot(h, w4_r[...], preferred_element_type=f32) + b4_r[...], 0.0)
    logit = (jnp.dot(g, wg_r[...], preferred_element_type=f32)
             + jnp.dot(h, wm_r[...], preferred_element_type=f32)
             + bo_r[...])
    o_r[...] = jax.nn.sigmoid(logit)[:, 0]


def _dense(u, i, w1a, w1b, b1, w2, b2, w3, b3, w4, b4, wg, wm, bo):
    B, D = u.shape
    BLK = 4096
    grid = (B // BLK,)

    def rowblk():
        return pl.BlockSpec((BLK, D), lambda k: (k, 0))

    def full(a):
        return pl.BlockSpec(a.shape, lambda k: (0,) * a.ndim)

    return pl.pallas_call(
        _dense_body,
        grid=grid,
        in_specs=[rowblk(), rowblk(),
                  full(w1a), full(w1b), full(b1), full(w2), full(b2),
                  full(w3), full(b3), full(w4), full(b4),
                  full(wg), full(wm), full(bo)],
        out_specs=pl.BlockSpec((BLK,), lambda k: (k,)),
        out_shape=jax.ShapeDtypeStruct((B,), jnp.float32),
    )(u, i, w1a, w1b, b1, w2, b2, w3, b3, w4, b4, wg, wm, bo)


def kernel(x, mlp_user_emb, mlp_item_emb, gmf_user_emb, gmf_item_emb,
           W1, b1, W2, b2, W3, b3, W4, b4, Wout, bout):
    V, P = mlp_user_emb.shape
    ut = jnp.concatenate([mlp_user_emb, gmf_user_emb], axis=1)
    it = jnp.concatenate([mlp_item_emb, gmf_item_emb], axis=1)
    uid = x[:, 0]
    iid = x[:, 1]
    gu, gi = _sc_gather(uid, iid, ut, it)
    return _dense(gu, gi,
                  W1[:P], W1[P:], b1[None, :], W2, b2[None, :],
                  W3, b3[None, :], W4, b4[None, :],
                  Wout[:P], Wout[P:], bout[None, :])
